# baseline (speedup 1.0000x reference)
import functools

import jax
import jax.numpy as jnp
from jax import lax
from jax.experimental import pallas as pl
from jax.experimental.pallas import tpu as pltpu
from jax.experimental.pallas import tpu_sc as plsc

_B = 1024
_D_NUM = 256
_F = 16
_D_TOK = 32
_N_CAT = 26
_CAT_SIZE = 100000

_N_WORKERS = 32
_ROWS = _N_CAT * _D_NUM
_R_PER_W = _ROWS // _N_WORKERS
_HALF = _R_PER_W // 2

def _sc_gather_body(idx_hbm, emb_hbm, out_hbm, idx_v, rows_v, sem):
    w = lax.axis_index("s") * 2 + lax.axis_index("c")
    base = w * _R_PER_W
    pltpu.sync_copy(idx_hbm.at[w], idx_v)
    cp0 = pltpu.async_copy(emb_hbm.at[idx_v.at[0]], rows_v.at[pl.ds(0, _HALF)], sem)
    cp1 = pltpu.async_copy(emb_hbm.at[idx_v.at[1]], rows_v.at[pl.ds(_HALF, _HALF)], sem)
    cp0.wait()
    cp1.wait()
    pltpu.sync_copy(rows_v, out_hbm.at[pl.ds(base, _R_PER_W)])


@functools.cache
def _make_sc_gather():
    mesh = plsc.VectorSubcoreMesh(
        core_axis_name="c", subcore_axis_name="s", num_cores=2, num_subcores=16
    )
    return pl.kernel(
        _sc_gather_body,
        out_type=jax.ShapeDtypeStruct((_ROWS, _D_TOK), jnp.float32),
        mesh=mesh,
        scratch_types=[
            pltpu.VMEM((2, _HALF), jnp.int32),
            pltpu.VMEM((_R_PER_W, _D_TOK), jnp.float32),
            pltpu.SemaphoreType.DMA,
        ],
        compiler_params=pltpu.CompilerParams(use_tc_tiling_on_sc=False),
    )


_TILE = 128
_GRID = 9


def _tc_body(x_ref, g_ref, w_ref, wq_ref, b_ref, out_ref):
    i = pl.program_id(0)

    @pl.when(i < _GRID - 1)
    def _dense():
        w = w_ref[...]
        out_ref[:, 0:_F, :] = jnp.broadcast_to(
            w[0:_F][None], (_TILE, _F, _D_TOK)
        )
        out_ref[:, _F:, :] = x_ref[...][:, :, None] * w[_F:][None] + b_ref[...][None]

    @pl.when(i == _GRID - 1)
    def _cat():
        out_ref[0:_N_CAT, 0:_F, :] = jnp.broadcast_to(
            wq_ref[...][None], (_N_CAT, _F, _D_TOK)
        )
        out_ref[0:_N_CAT, _F:, :] = g_ref[...] + b_ref[...][None]


def kernel(x_num, x_cat, emb, weight_, weight, bias):
    offsets = jnp.arange(_N_CAT, dtype=jnp.int32) * _CAT_SIZE
    idx3 = (x_cat + offsets[:, None]).reshape(_N_WORKERS, 2, _HALF)
    g = _make_sc_gather()(idx3, emb)
    g3 = g.reshape(_N_CAT, _D_NUM, _D_TOK)
    return pl.pallas_call(
        _tc_body,
        grid=(_GRID,),
        in_specs=[
            pl.BlockSpec((_TILE, _D_NUM), lambda i: (jnp.minimum(i, _GRID - 2), 0)),
            pl.BlockSpec((_N_CAT, _D_NUM, _D_TOK), lambda i: (0, 0, 0)),
            pl.BlockSpec((_F + _D_NUM, _D_TOK), lambda i: (0, 0)),
            pl.BlockSpec((_F, _D_TOK), lambda i: (0, 0)),
            pl.BlockSpec((_D_NUM, _D_TOK), lambda i: (0, 0)),
        ],
        out_specs=pl.BlockSpec((_TILE, _F + _D_NUM, _D_TOK), lambda i: (i, 0, 0)),
        out_shape=jax.ShapeDtypeStruct((_B + _N_CAT, _F + _D_NUM, _D_TOK), jnp.float32),
    )(x_num, g3, weight, weight_, bias)
